# baseline (speedup 1.0000x reference)
import functools

import jax
import jax.numpy as jnp
from jax import lax
from jax.experimental import pallas as pl
from jax.experimental.pallas import tpu as pltpu
from jax.experimental.pallas import tpu_sc as plsc

_B = 16384
_F = 8

_info = plsc.get_sparse_core_info()
_NC, _NS = _info.num_cores, _info.num_subcores
_NW = _NC * _NS
_G = 16
_SLAB = 128
_NBUF = 4


def _make_sc_gather(nb):
    bpw = nb // _NW
    ng = bpw // _G
    nblk = bpw // 128

    def body(ut_ref, tt_ref, users_ref, tracks_ref, u_out, t_out,
             idx_v, slabs, outb, sems):
        wid = lax.axis_index("s") * _NC + lax.axis_index("c")
        base = wid * bpw

        def fire(table, g, slot):
            iv = idx_v[pl.ds(g * _G, _G)]
            tv = lax.shift_left(lax.shift_right_logical(iv, 7), 7)
            for s in range(_G):
                col0 = pl.multiple_of(tv[s], 128)
                pltpu.async_copy(
                    table.at[:, pl.ds(col0, _SLAB)], slabs.at[slot, s],
                    sems.at[slot])

        def drain(out_hbm, slot):
            pltpu.make_async_copy(
                out_hbm.at[pl.ds(0, _G)], slabs.at[slot, pl.ds(0, _G)],
                sems.at[slot]).wait()

        lanes = lax.iota(jnp.int32, _G)
        masks = [lanes == s for s in range(_G)]
        zero = jnp.zeros((_G,), jnp.float32)

        def extract(g, slot):
            cv = idx_v[pl.ds(g * _G, _G)] & (_SLAB - 1)
            offs = [cv[s] - s for s in range(_G)]
            blk = lax.shift_right_logical(g, 3)
            lane0 = lax.shift_left(g & 7, 4)
            for f in range(_F):
                parts = [
                    jnp.where(masks[s], slabs[slot, s, f, pl.ds(offs[s], _G)],
                              zero)
                    for s in range(_G)
                ]
                while len(parts) > 1:
                    parts = [a + b for a, b in zip(parts[::2], parts[1::2])]
                outb[blk, f, pl.ds(lane0, _G)] = parts[0]

        def do_table(table, idx_hbm, out_hbm):
            pltpu.sync_copy(idx_hbm.at[pl.ds(base, bpw)], idx_v)
            for j in range(_NBUF):
                fire(table, j, j)

            def step(k, _):
                for j in range(_NBUF):
                    g = _NBUF * k + j
                    drain(out_hbm, j)
                    extract(g, j)

                    @pl.when(g + _NBUF < ng)
                    def _():
                        fire(table, g + _NBUF, j)
                return ()

            lax.fori_loop(0, ng // _NBUF, step, (), unroll=False)
            for b in range(nblk):
                pltpu.sync_copy(outb.at[b], out_hbm.at[wid * nblk + b])

        do_table(ut_ref, users_ref, u_out)
        do_table(tt_ref, tracks_ref, t_out)

    return pl.kernel(
        body,
        mesh=plsc.VectorSubcoreMesh(core_axis_name="c", subcore_axis_name="s"),
        out_type=[
            jax.ShapeDtypeStruct((nb // 128, _F, 128), jnp.float32),
            jax.ShapeDtypeStruct((nb // 128, _F, 128), jnp.float32),
        ],
        scratch_types=[
            pltpu.VMEM((bpw,), jnp.int32),
            pltpu.VMEM((_NBUF, _G + 1, _F, _SLAB), jnp.float32),
            pltpu.VMEM((nblk, _F, 128), jnp.float32),
            pltpu.SemaphoreType.DMA((_NBUF,)),
        ],
    )


def _mlp_body(u_ref, t_ref, w1_ref, b1_ref, w2_ref, b2_ref, w3_ref, b3_ref,
              o_ref):
    dn = (((0,), (0,)), ((), ()))
    h = lax.dot_general(w1_ref[0:_F, :], u_ref[...], dn,
                        preferred_element_type=jnp.float32)
    h = h + lax.dot_general(w1_ref[_F:2 * _F, :], t_ref[...], dn,
                            preferred_element_type=jnp.float32)
    h = jnp.maximum(h + b1_ref[...], 0.0)
    h = lax.dot_general(w2_ref[...], h, dn, preferred_element_type=jnp.float32)
    h = jnp.maximum(h + b2_ref[...], 0.0)
    o = lax.dot_general(w3_ref[...], h, dn,
                        preferred_element_type=jnp.float32) + b3_ref[...]
    o_ref[...] = 1.0 / (1.0 + jnp.exp(-o))


def _mlp(u2, t2, W1, b1c, W2, b2c, W3, b3c, nb):
    bn = 2048
    return pl.pallas_call(
        _mlp_body,
        grid=(nb // bn,),
        in_specs=[
            pl.BlockSpec((_F, bn), lambda i: (0, i)),
            pl.BlockSpec((_F, bn), lambda i: (0, i)),
            pl.BlockSpec((2 * _F, 64), lambda i: (0, 0)),
            pl.BlockSpec((64, 1), lambda i: (0, 0)),
            pl.BlockSpec((64, 32), lambda i: (0, 0)),
            pl.BlockSpec((32, 1), lambda i: (0, 0)),
            pl.BlockSpec((32, 1), lambda i: (0, 0)),
            pl.BlockSpec((1, 1), lambda i: (0, 0)),
        ],
        out_specs=pl.BlockSpec((1, bn), lambda i: (0, i)),
        out_shape=jax.ShapeDtypeStruct((1, nb), jnp.float32),
    )(u2, t2, W1, b1c, W2, b2c, W3, b3c)


def kernel(users, tracks, user_table, track_table, W1, b1, W2, b2, W3, b3):
    ut = user_table.T
    tt = track_table.T
    b1c = b1.reshape(64, 1)
    b2c = b2.reshape(32, 1)
    b3c = b3.reshape(1, 1)
    u_emb, t_emb = _make_sc_gather(_B)(ut, tt, users, tracks)
    u2 = jnp.transpose(u_emb, (1, 0, 2)).reshape(_F, _B)
    t2 = jnp.transpose(t_emb, (1, 0, 2)).reshape(_F, _B)
    return _mlp(u2, t2, W1, b1c, W2, b2c, W3, b3c, _B).reshape(_B, 1)


# speedup vs baseline: 1.0054x; 1.0054x over previous
import functools

import jax
import jax.numpy as jnp
from jax import lax
from jax.experimental import pallas as pl
from jax.experimental.pallas import tpu as pltpu
from jax.experimental.pallas import tpu_sc as plsc

_B = 16384
_F = 8

_info = plsc.get_sparse_core_info()
_NC, _NS = _info.num_cores, _info.num_subcores
_NW = _NC * _NS
_G = 16
_SLAB = 128
_NBUF = 4


def _make_sc_gather(nb):
    bpw = nb // _NW
    ng = bpw // _G
    nblk = bpw // 128

    def body(ut_ref, tt_ref, users_ref, tracks_ref, u_out, t_out,
             idx_v, slabs, outb, sems):
        wid = lax.axis_index("s") * _NC + lax.axis_index("c")
        base = wid * bpw

        def fire(table, g, slot):
            iv = idx_v[pl.ds(g * _G, _G)]
            tv = lax.shift_left(lax.shift_right_logical(iv, 7), 7)
            for s in range(_G):
                col0 = pl.multiple_of(tv[s], 128)
                pltpu.async_copy(
                    table.at[:, pl.ds(col0, _SLAB)], slabs.at[slot, s],
                    sems.at[slot])

        def drain(out_hbm, slot):
            pltpu.make_async_copy(
                out_hbm.at[pl.ds(0, _G)], slabs.at[slot, pl.ds(0, _G)],
                sems.at[slot]).wait()

        lanes = lax.iota(jnp.int32, _G)
        bitmasks = [(lanes & (1 << l)) != 0 for l in range(4)]

        def extract(g, slot):
            cv = idx_v[pl.ds(g * _G, _G)] & (_SLAB - 1)
            offs = [cv[s] - s for s in range(_G)]
            blk = lax.shift_right_logical(g, 3)
            lane0 = lax.shift_left(g & 7, 4)
            for f in range(_F):
                parts = [slabs[slot, s, f, pl.ds(offs[s], _G)]
                         for s in range(_G)]
                for l in range(4):
                    parts = [jnp.where(bitmasks[l], hi, lo)
                             for lo, hi in zip(parts[::2], parts[1::2])]
                outb[blk, f, pl.ds(lane0, _G)] = parts[0]

        def do_table(table, idx_hbm, out_hbm):
            pltpu.sync_copy(idx_hbm.at[pl.ds(base, bpw)], idx_v)
            for j in range(_NBUF):
                fire(table, j, j)

            def step(k, _):
                for j in range(_NBUF):
                    g = _NBUF * k + j
                    drain(out_hbm, j)
                    extract(g, j)

                    @pl.when(g + _NBUF < ng)
                    def _():
                        fire(table, g + _NBUF, j)
                return ()

            lax.fori_loop(0, ng // _NBUF, step, (), unroll=False)
            for b in range(nblk):
                pltpu.sync_copy(outb.at[b], out_hbm.at[wid * nblk + b])

        do_table(ut_ref, users_ref, u_out)
        do_table(tt_ref, tracks_ref, t_out)

    return pl.kernel(
        body,
        mesh=plsc.VectorSubcoreMesh(core_axis_name="c", subcore_axis_name="s"),
        out_type=[
            jax.ShapeDtypeStruct((nb // 128, _F, 128), jnp.float32),
            jax.ShapeDtypeStruct((nb // 128, _F, 128), jnp.float32),
        ],
        scratch_types=[
            pltpu.VMEM((bpw,), jnp.int32),
            pltpu.VMEM((_NBUF, _G + 1, _F, _SLAB), jnp.float32),
            pltpu.VMEM((nblk, _F, 128), jnp.float32),
            pltpu.SemaphoreType.DMA((_NBUF,)),
        ],
    )


def _mlp_body(u_ref, t_ref, w1_ref, b1_ref, w2_ref, b2_ref, w3_ref, b3_ref,
              o_ref):
    dn = (((0,), (0,)), ((), ()))
    h = lax.dot_general(w1_ref[0:_F, :], u_ref[...], dn,
                        preferred_element_type=jnp.float32)
    h = h + lax.dot_general(w1_ref[_F:2 * _F, :], t_ref[...], dn,
                            preferred_element_type=jnp.float32)
    h = jnp.maximum(h + b1_ref[...], 0.0)
    h = lax.dot_general(w2_ref[...], h, dn, preferred_element_type=jnp.float32)
    h = jnp.maximum(h + b2_ref[...], 0.0)
    o = lax.dot_general(w3_ref[...], h, dn,
                        preferred_element_type=jnp.float32) + b3_ref[...]
    o_ref[...] = 1.0 / (1.0 + jnp.exp(-o))


def _mlp(u2, t2, W1, b1c, W2, b2c, W3, b3c, nb):
    bn = 2048
    return pl.pallas_call(
        _mlp_body,
        grid=(nb // bn,),
        in_specs=[
            pl.BlockSpec((_F, bn), lambda i: (0, i)),
            pl.BlockSpec((_F, bn), lambda i: (0, i)),
            pl.BlockSpec((2 * _F, 64), lambda i: (0, 0)),
            pl.BlockSpec((64, 1), lambda i: (0, 0)),
            pl.BlockSpec((64, 32), lambda i: (0, 0)),
            pl.BlockSpec((32, 1), lambda i: (0, 0)),
            pl.BlockSpec((32, 1), lambda i: (0, 0)),
            pl.BlockSpec((1, 1), lambda i: (0, 0)),
        ],
        out_specs=pl.BlockSpec((1, bn), lambda i: (0, i)),
        out_shape=jax.ShapeDtypeStruct((1, nb), jnp.float32),
    )(u2, t2, W1, b1c, W2, b2c, W3, b3c)


def kernel(users, tracks, user_table, track_table, W1, b1, W2, b2, W3, b3):
    ut = user_table.T
    tt = track_table.T
    b1c = b1.reshape(64, 1)
    b2c = b2.reshape(32, 1)
    b3c = b3.reshape(1, 1)
    u_emb, t_emb = _make_sc_gather(_B)(ut, tt, users, tracks)
    u2 = jnp.transpose(u_emb, (1, 0, 2)).reshape(_F, _B)
    t2 = jnp.transpose(t_emb, (1, 0, 2)).reshape(_F, _B)
    return _mlp(u2, t2, W1, b1c, W2, b2c, W3, b3c, _B).reshape(_B, 1)


# speedup vs baseline: 1.0412x; 1.0356x over previous
import functools

import jax
import jax.numpy as jnp
from jax import lax
from jax.experimental import pallas as pl
from jax.experimental.pallas import tpu as pltpu
from jax.experimental.pallas import tpu_sc as plsc

_B = 16384
_F = 8

_info = plsc.get_sparse_core_info()
_NC, _NS = _info.num_cores, _info.num_subcores
_NW = _NC * _NS
_G = 16
_SLAB = 128
_NBUF = 4


def _make_sc_gather(nb):
    bpw = nb // _NW
    ng = bpw // _G
    nblk = bpw // 128

    def body(ut_ref, tt_ref, users_ref, tracks_ref, u_out, t_out,
             idx_v, slabs, outb, sems):
        wid = lax.axis_index("s") * _NC + lax.axis_index("c")
        base = wid * bpw

        def fire(table, g, slot):
            iv = idx_v[pl.ds(g * _G, _G)]
            tv = lax.shift_left(lax.shift_right_logical(iv, 7), 7)
            for s in range(_G):
                col0 = pl.multiple_of(tv[s], 128)
                pltpu.async_copy(
                    table.at[:, pl.ds(col0, _SLAB)], slabs.at[slot, s],
                    sems.at[slot])

        def drain(out_hbm, slot):
            pltpu.make_async_copy(
                out_hbm.at[pl.ds(0, _G)], slabs.at[slot, pl.ds(0, _G)],
                sems.at[slot]).wait()

        lanes = lax.iota(jnp.int32, _G)
        bitmasks = [(lanes & (1 << l)) != 0 for l in range(4)]

        def extract(g, slot):
            cv = idx_v[pl.ds(g * _G, _G)] & (_SLAB - 1)
            offs = [cv[s] - s for s in range(_G)]
            blk = lax.shift_right_logical(g, 3)
            lane0 = lax.shift_left(g & 7, 4)
            for f in range(_F):
                parts = [slabs[slot, s, f, pl.ds(offs[s], _G)]
                         for s in range(_G)]
                for l in range(4):
                    parts = [jnp.where(bitmasks[l], hi, lo)
                             for lo, hi in zip(parts[::2], parts[1::2])]
                outb[blk, f, pl.ds(lane0, _G)] = parts[0]

        def do_table(table, idx_hbm, out_hbm):
            pltpu.sync_copy(idx_hbm.at[pl.ds(base, bpw)], idx_v)
            for j in range(_NBUF):
                fire(table, j, j)

            def step(k, _):
                for j in range(_NBUF):
                    g = _NBUF * k + j
                    drain(out_hbm, j)
                    extract(g, j)

                    @pl.when(g + _NBUF < ng)
                    def _():
                        fire(table, g + _NBUF, j)
                return ()

            lax.fori_loop(0, ng // _NBUF, step, (), unroll=False)
            for b in range(nblk):
                pltpu.sync_copy(outb.at[b], out_hbm.at[wid * nblk + b])

        do_table(ut_ref, users_ref, u_out)
        do_table(tt_ref, tracks_ref, t_out)

    return pl.kernel(
        body,
        mesh=plsc.VectorSubcoreMesh(core_axis_name="c", subcore_axis_name="s"),
        out_type=[
            jax.ShapeDtypeStruct((nb // 128, _F, 128), jnp.float32),
            jax.ShapeDtypeStruct((nb // 128, _F, 128), jnp.float32),
        ],
        scratch_types=[
            pltpu.VMEM((bpw,), jnp.int32),
            pltpu.VMEM((_NBUF, _G + 1, _F, _SLAB), jnp.float32),
            pltpu.VMEM((nblk, _F, 128), jnp.float32),
            pltpu.SemaphoreType.DMA((_NBUF,)),
        ],
    )


def _mlp_body(u_ref, t_ref, w1_ref, b1_ref, w2_ref, b2_ref, w3_ref, b3_ref,
              o_ref):
    dn = (((0,), (0,)), ((), ()))
    h = lax.dot_general(w1_ref[0:_F, :], u_ref[...], dn,
                        preferred_element_type=jnp.float32)
    h = h + lax.dot_general(w1_ref[_F:2 * _F, :], t_ref[...], dn,
                            preferred_element_type=jnp.float32)
    h = jnp.maximum(h + b1_ref[...], 0.0)
    h = lax.dot_general(w2_ref[...], h, dn, preferred_element_type=jnp.float32)
    h = jnp.maximum(h + b2_ref[...], 0.0)
    o = lax.dot_general(w3_ref[...], h, dn,
                        preferred_element_type=jnp.float32) + b3_ref[...]
    o_ref[...] = 1.0 / (1.0 + jnp.exp(-o))


def _mlp(u2, t2, W1, b1c, W2, b2c, W3, b3c, nb):
    bn = 4096
    return pl.pallas_call(
        _mlp_body,
        grid=(nb // bn,),
        in_specs=[
            pl.BlockSpec((_F, bn), lambda i: (0, i)),
            pl.BlockSpec((_F, bn), lambda i: (0, i)),
            pl.BlockSpec((2 * _F, 64), lambda i: (0, 0)),
            pl.BlockSpec((64, 1), lambda i: (0, 0)),
            pl.BlockSpec((64, 32), lambda i: (0, 0)),
            pl.BlockSpec((32, 1), lambda i: (0, 0)),
            pl.BlockSpec((32, 1), lambda i: (0, 0)),
            pl.BlockSpec((1, 1), lambda i: (0, 0)),
        ],
        out_specs=pl.BlockSpec((1, bn), lambda i: (0, i)),
        out_shape=jax.ShapeDtypeStruct((1, nb), jnp.float32),
    )(u2, t2, W1, b1c, W2, b2c, W3, b3c)


def kernel(users, tracks, user_table, track_table, W1, b1, W2, b2, W3, b3):
    ut = user_table.T
    tt = track_table.T
    b1c = b1.reshape(64, 1)
    b2c = b2.reshape(32, 1)
    b3c = b3.reshape(1, 1)
    u_emb, t_emb = _make_sc_gather(_B)(ut, tt, users, tracks)
    u2 = jnp.transpose(u_emb, (1, 0, 2)).reshape(_F, _B)
    t2 = jnp.transpose(t_emb, (1, 0, 2)).reshape(_F, _B)
    return _mlp(u2, t2, W1, b1c, W2, b2c, W3, b3c, _B).reshape(_B, 1)


# speedup vs baseline: 1.0571x; 1.0153x over previous
import functools

import jax
import jax.numpy as jnp
from jax import lax
from jax.experimental import pallas as pl
from jax.experimental.pallas import tpu as pltpu
from jax.experimental.pallas import tpu_sc as plsc

_B = 16384
_F = 8

_info = plsc.get_sparse_core_info()
_NC, _NS = _info.num_cores, _info.num_subcores
_NW = _NC * _NS
_G = 16
_SLAB = 128
_NBUF = 4


def _make_sc_gather(nb):
    bpw = nb // _NW
    ng = bpw // _G
    nblk = bpw // 128

    def body(ut_ref, tt_ref, users_ref, tracks_ref, u_out, t_out,
             idx_v, slabs, outb, sems):
        wid = lax.axis_index("s") * _NC + lax.axis_index("c")
        base = wid * bpw

        def fire(table, g, slot):
            iv = idx_v[pl.ds(g * _G, _G)]
            tv = lax.shift_left(lax.shift_right_logical(iv, 7), 7)
            for s in range(_G):
                col0 = pl.multiple_of(tv[s], 128)
                pltpu.async_copy(
                    table.at[:, pl.ds(col0, _SLAB)], slabs.at[slot, s],
                    sems.at[slot])

        def drain(out_hbm, slot):
            pltpu.make_async_copy(
                out_hbm.at[pl.ds(0, _G)], slabs.at[slot, pl.ds(0, _G)],
                sems.at[slot]).wait()

        lanes = lax.iota(jnp.int32, _G)
        bitmasks = [(lanes & (1 << l)) != 0 for l in range(4)]

        def extract(g, slot):
            cv = idx_v[pl.ds(g * _G, _G)] & (_SLAB - 1)
            offs = [cv[s] - s for s in range(_G)]
            blk = lax.shift_right_logical(g, 3)
            lane0 = lax.shift_left(g & 7, 4)
            for f in range(_F):
                parts = [slabs[slot, s, f, pl.ds(offs[s], _G)]
                         for s in range(_G)]
                for l in range(4):
                    parts = [jnp.where(bitmasks[l], hi, lo)
                             for lo, hi in zip(parts[::2], parts[1::2])]
                outb[blk, f, pl.ds(lane0, _G)] = parts[0]

        def do_table(table, idx_hbm, out_hbm):
            pltpu.sync_copy(idx_hbm.at[pl.ds(base, bpw)], idx_v)
            for j in range(_NBUF):
                fire(table, j, j)

            def step(k, _):
                for j in range(_NBUF):
                    g = _NBUF * k + j
                    drain(out_hbm, j)
                    extract(g, j)

                    @pl.when(g + _NBUF < ng)
                    def _():
                        fire(table, g + _NBUF, j)
                return ()

            lax.fori_loop(0, ng // _NBUF, step, (), unroll=False)
            for b in range(nblk):
                pltpu.sync_copy(outb.at[b], out_hbm.at[wid * nblk + b])

        do_table(ut_ref, users_ref, u_out)
        do_table(tt_ref, tracks_ref, t_out)

    return pl.kernel(
        body,
        mesh=plsc.VectorSubcoreMesh(core_axis_name="c", subcore_axis_name="s"),
        out_type=[
            jax.ShapeDtypeStruct((nb // 128, _F, 128), jnp.float32),
            jax.ShapeDtypeStruct((nb // 128, _F, 128), jnp.float32),
        ],
        scratch_types=[
            pltpu.VMEM((bpw,), jnp.int32),
            pltpu.VMEM((_NBUF, _G + 1, _F, _SLAB), jnp.float32),
            pltpu.VMEM((nblk, _F, 128), jnp.float32),
            pltpu.SemaphoreType.DMA((_NBUF,)),
        ],
    )


def _mlp_body(u_ref, t_ref, w1_ref, b1_ref, w2_ref, b2_ref, w3_ref, b3_ref,
              o_ref):
    dn = (((0,), (0,)), ((), ()))
    h = lax.dot_general(w1_ref[0:_F, :], u_ref[...], dn,
                        preferred_element_type=jnp.float32)
    h = h + lax.dot_general(w1_ref[_F:2 * _F, :], t_ref[...], dn,
                            preferred_element_type=jnp.float32)
    h = jnp.maximum(h + b1_ref[...], 0.0)
    h = lax.dot_general(w2_ref[...], h, dn, preferred_element_type=jnp.float32)
    h = jnp.maximum(h + b2_ref[...], 0.0)
    o = lax.dot_general(w3_ref[...], h, dn,
                        preferred_element_type=jnp.float32) + b3_ref[...]
    o_ref[...] = 1.0 / (1.0 + jnp.exp(-o))


def _mlp(u2, t2, W1, b1c, W2, b2c, W3, b3c, nb):
    bn = 16384
    return pl.pallas_call(
        _mlp_body,
        grid=(nb // bn,),
        in_specs=[
            pl.BlockSpec((_F, bn), lambda i: (0, i)),
            pl.BlockSpec((_F, bn), lambda i: (0, i)),
            pl.BlockSpec((2 * _F, 64), lambda i: (0, 0)),
            pl.BlockSpec((64, 1), lambda i: (0, 0)),
            pl.BlockSpec((64, 32), lambda i: (0, 0)),
            pl.BlockSpec((32, 1), lambda i: (0, 0)),
            pl.BlockSpec((32, 1), lambda i: (0, 0)),
            pl.BlockSpec((1, 1), lambda i: (0, 0)),
        ],
        out_specs=pl.BlockSpec((1, bn), lambda i: (0, i)),
        out_shape=jax.ShapeDtypeStruct((1, nb), jnp.float32),
    )(u2, t2, W1, b1c, W2, b2c, W3, b3c)


def kernel(users, tracks, user_table, track_table, W1, b1, W2, b2, W3, b3):
    ut = user_table.T
    tt = track_table.T
    b1c = b1.reshape(64, 1)
    b2c = b2.reshape(32, 1)
    b3c = b3.reshape(1, 1)
    u_emb, t_emb = _make_sc_gather(_B)(ut, tt, users, tracks)
    u2 = jnp.transpose(u_emb, (1, 0, 2)).reshape(_F, _B)
    t2 = jnp.transpose(t_emb, (1, 0, 2)).reshape(_F, _B)
    return _mlp(u2, t2, W1, b1c, W2, b2c, W3, b3c, _B).reshape(_B, 1)


# speedup vs baseline: 1.1032x; 1.0436x over previous
import functools

import jax
import jax.numpy as jnp
from jax import lax
from jax.experimental import pallas as pl
from jax.experimental.pallas import tpu as pltpu
from jax.experimental.pallas import tpu_sc as plsc

_B = 16384
_F = 8

_info = plsc.get_sparse_core_info()
_NC, _NS = _info.num_cores, _info.num_subcores
_NW = _NC * _NS
_G = 16
_SLAB = 128
_NBUF = 4


def _make_sc_gather(nb):
    bpw = nb // _NW
    ng = bpw // _G
    nblk = bpw // 128

    def body(ut_ref, tt_ref, users_ref, tracks_ref, u_out, t_out,
             idx_v, slabs, outb, sems):
        wid = lax.axis_index("s") * _NC + lax.axis_index("c")
        base = wid * bpw

        def fire(table, g, slot):
            iv = idx_v[slot // 2, pl.ds(g * _G, _G)]
            tv = lax.shift_left(lax.shift_right_logical(iv, 7), 7)
            for s in range(_G):
                col0 = pl.multiple_of(tv[s], 128)
                pltpu.async_copy(
                    table.at[:, pl.ds(col0, _SLAB)], slabs.at[slot, s],
                    sems.at[slot])

        def drain(out_hbm, slot):
            pltpu.make_async_copy(
                out_hbm.at[pl.ds(0, _G)], slabs.at[slot, pl.ds(0, _G)],
                sems.at[slot]).wait()

        lanes = lax.iota(jnp.int32, _G)
        bitmasks = [(lanes & (1 << l)) != 0 for l in range(4)]

        def extract(g, slot):
            cv = idx_v[slot // 2, pl.ds(g * _G, _G)] & (_SLAB - 1)
            offs = [cv[s] - s for s in range(_G)]
            blk = lax.shift_right_logical(g, 3)
            lane0 = lax.shift_left(g & 7, 4)
            for f in range(_F):
                parts = [slabs[slot, s, f, pl.ds(offs[s], _G)]
                         for s in range(_G)]
                for l in range(4):
                    parts = [jnp.where(bitmasks[l], hi, lo)
                             for lo, hi in zip(parts[::2], parts[1::2])]
                outb[slot // 2, blk, f, pl.ds(lane0, _G)] = parts[0]

        half = _NBUF // 2
        plan = [(ut_ref, u_out, 0, 0), (ut_ref, u_out, 1, 1),
                (tt_ref, t_out, 2, 0), (tt_ref, t_out, 3, 1)]

        pltpu.sync_copy(users_ref.at[pl.ds(base, bpw)], idx_v.at[0])
        pltpu.sync_copy(tracks_ref.at[pl.ds(base, bpw)], idx_v.at[1])
        for table, _, slot, goff in plan:
            fire(table, goff, slot)

        def step(k, _):
            for table, out_hbm, slot, goff in plan:
                g = half * k + goff
                drain(out_hbm, slot)
                extract(g, slot)

                @pl.when(g + half < ng)
                def _():
                    fire(table, g + half, slot)
            return ()

        lax.fori_loop(0, ng // half, step, (), unroll=False)
        for b in range(nblk):
            pltpu.sync_copy(outb.at[0, b], u_out.at[wid * nblk + b])
            pltpu.sync_copy(outb.at[1, b], t_out.at[wid * nblk + b])

    return pl.kernel(
        body,
        mesh=plsc.VectorSubcoreMesh(core_axis_name="c", subcore_axis_name="s"),
        out_type=[
            jax.ShapeDtypeStruct((nb // 128, _F, 128), jnp.float32),
            jax.ShapeDtypeStruct((nb // 128, _F, 128), jnp.float32),
        ],
        scratch_types=[
            pltpu.VMEM((2, bpw), jnp.int32),
            pltpu.VMEM((_NBUF, _G + 1, _F, _SLAB), jnp.float32),
            pltpu.VMEM((2, nblk, _F, 128), jnp.float32),
            pltpu.SemaphoreType.DMA((_NBUF,)),
        ],
    )


def _mlp_body(u_ref, t_ref, w1_ref, b1_ref, w2_ref, b2_ref, w3_ref, b3_ref,
              o_ref):
    dn = (((0,), (0,)), ((), ()))
    h = lax.dot_general(w1_ref[0:_F, :], u_ref[...], dn,
                        preferred_element_type=jnp.float32)
    h = h + lax.dot_general(w1_ref[_F:2 * _F, :], t_ref[...], dn,
                            preferred_element_type=jnp.float32)
    h = jnp.maximum(h + b1_ref[...], 0.0)
    h = lax.dot_general(w2_ref[...], h, dn, preferred_element_type=jnp.float32)
    h = jnp.maximum(h + b2_ref[...], 0.0)
    o = lax.dot_general(w3_ref[...], h, dn,
                        preferred_element_type=jnp.float32) + b3_ref[...]
    o_ref[...] = 1.0 / (1.0 + jnp.exp(-o))


def _mlp(u2, t2, W1, b1c, W2, b2c, W3, b3c, nb):
    bn = 16384
    return pl.pallas_call(
        _mlp_body,
        grid=(nb // bn,),
        in_specs=[
            pl.BlockSpec((_F, bn), lambda i: (0, i)),
            pl.BlockSpec((_F, bn), lambda i: (0, i)),
            pl.BlockSpec((2 * _F, 64), lambda i: (0, 0)),
            pl.BlockSpec((64, 1), lambda i: (0, 0)),
            pl.BlockSpec((64, 32), lambda i: (0, 0)),
            pl.BlockSpec((32, 1), lambda i: (0, 0)),
            pl.BlockSpec((32, 1), lambda i: (0, 0)),
            pl.BlockSpec((1, 1), lambda i: (0, 0)),
        ],
        out_specs=pl.BlockSpec((1, bn), lambda i: (0, i)),
        out_shape=jax.ShapeDtypeStruct((1, nb), jnp.float32),
    )(u2, t2, W1, b1c, W2, b2c, W3, b3c)


def kernel(users, tracks, user_table, track_table, W1, b1, W2, b2, W3, b3):
    ut = user_table.T
    tt = track_table.T
    b1c = b1.reshape(64, 1)
    b2c = b2.reshape(32, 1)
    b3c = b3.reshape(1, 1)
    u_emb, t_emb = _make_sc_gather(_B)(ut, tt, users, tracks)
    u2 = jnp.transpose(u_emb, (1, 0, 2)).reshape(_F, _B)
    t2 = jnp.transpose(t_emb, (1, 0, 2)).reshape(_F, _B)
    return _mlp(u2, t2, W1, b1c, W2, b2c, W3, b3c, _B).reshape(_B, 1)
